# Initial kernel scaffold; baseline (speedup 1.0000x reference)
import jax
import jax.numpy as jnp
from jax.experimental import pallas as pl


def kernel(x, batch, W1, b1, g1, be1, W2, b2, g2, be2, W3, b3, g3, be3, Wg1, bg1, gg1, beg1, Wg2, bg2, gg2, beg2, Wb1, bb1, gb1, beb1, Wb2, bb2):
    raise NotImplementedError("write your pallas kernel here")


# speedup vs baseline: 2.8938x; 2.8938x over previous
import jax
import jax.numpy as jnp
from jax import lax
from jax.experimental import pallas as pl
from jax.experimental.pallas import tpu as pltpu

B = 8
P = 2048
K = 16
RB = 256
NRB = P // RB
N = B * P

_BIG = 1e10
_INF = float("inf")
_NEG = -3.0e38
_PREC = None
_GPREC = lax.Precision.HIGHEST


def _topk_selectors(dist):
    col = lax.broadcasted_iota(jnp.int32, dist.shape, 1)
    d = dist
    for _ in range(K):
        m = jnp.min(d, axis=1, keepdims=True)
        cand = jnp.where(d == m, col, P)
        a = jnp.min(cand, axis=1, keepdims=True)
        sel = col == a
        d = jnp.where(sel, _INF, d)
        yield sel.astype(jnp.float32)


def _dist_block(hr, hc, r):
    d2c = jnp.sum(hc * hc, axis=1)
    d2r = jnp.sum(hr * hr, axis=1)
    g = lax.dot_general(hr, hc, (((1,), (1,)), ((), ())),
                        preferred_element_type=jnp.float32,
                        precision=_PREC)
    dist = d2r[:, None] + d2c[None, :] - 2.0 * g
    row = lax.broadcasted_iota(jnp.int32, (RB, P), 0)
    col = lax.broadcasted_iota(jnp.int32, (RB, P), 1)
    return jnp.where(col == row + r * RB, dist + _BIG, dist)


def _tree_sum(terms):
    while len(terms) > 1:
        terms = [terms[i] + terms[i + 1] for i in range(0, len(terms), 2)]
    return terms[0]


def _sublane_sum(terms):
    t = [terms[k] + terms[k + 8] for k in range(8)]
    t = [t[k] + t[k + 4] for k in range(4)]
    t = [t[k] + t[k + 2] for k in range(2)]
    return t[0] + t[1]


def _knn_cov_body(xc_ref, xr_ref, out_ref):
    r = pl.program_id(1)
    xc = xc_ref[0]
    xr = xr_ref[0]
    dist = _dist_block(xr, xc, r)
    nbs = [lax.dot_general(sel, xc, (((1,), (0,)), ((), ())),
                           preferred_element_type=jnp.float32,
                           precision=_GPREC)
           for sel in _topk_selectors(dist)]
    mean = _sublane_sum(nbs) * (1.0 / K)
    cs = [(nb - mean).astype(jnp.bfloat16).astype(jnp.float32) for nb in nbs]
    covs = []
    for i in range(3):
        for j in range(3):
            acc = _tree_sum([ck[:, i:i + 1] * ck[:, j:j + 1] for ck in cs])
            covs.append(acc * (1.0 / K))
    out_ref[0] = jnp.concatenate([xr] + covs, axis=1)


def _knn_cov(x3):
    return pl.pallas_call(
        _knn_cov_body,
        grid=(B, NRB),
        in_specs=[
            pl.BlockSpec((1, P, 3), lambda c, r: (c, 0, 0)),
            pl.BlockSpec((1, RB, 3), lambda c, r: (c, r, 0)),
        ],
        out_specs=pl.BlockSpec((1, RB, 12), lambda c, r: (c, r, 0)),
        out_shape=jax.ShapeDtypeStruct((B, P, 12), jnp.float32),
    )(x3, x3)


def _knn_pool_body(hc_ref, hr_ref, out_ref):
    r = pl.program_id(1)
    hc = hc_ref[0]
    hr = hr_ref[0]
    dist = _dist_block(hr, hc, r)
    acc = jnp.full((RB, hc.shape[1]), _NEG, jnp.float32)
    for sel in _topk_selectors(dist):
        nb = lax.dot_general(sel, hc, (((1,), (0,)), ((), ())),
                             preferred_element_type=jnp.float32,
                             precision=_GPREC)
        acc = jnp.maximum(acc, nb)
    out_ref[0] = acc


def _knn_pool(h, F):
    hc = h.reshape(B, P, F)
    out = pl.pallas_call(
        _knn_pool_body,
        grid=(B, NRB),
        in_specs=[
            pl.BlockSpec((1, P, F), lambda c, r: (c, 0, 0)),
            pl.BlockSpec((1, RB, F), lambda c, r: (c, r, 0)),
        ],
        out_specs=pl.BlockSpec((1, RB, F), lambda c, r: (c, r, 0)),
        out_shape=jax.ShapeDtypeStruct((B, P, F), jnp.float32),
    )(hc, hc)
    return out.reshape(N, F)


def _matmul_body(h_ref, W_ref, b_ref, out_ref):
    out_ref[...] = jnp.dot(h_ref[...], W_ref[...],
                           preferred_element_type=jnp.float32,
                           precision=_PREC) + b_ref[...][None, :]


def _matmul(h, W, b, rows_per_block=2048):
    n, fin = h.shape
    fout = W.shape[1]
    if n <= rows_per_block:
        return pl.pallas_call(
            _matmul_body,
            out_shape=jax.ShapeDtypeStruct((n, fout), jnp.float32),
        )(h, W, b)
    nb = n // rows_per_block
    return pl.pallas_call(
        _matmul_body,
        grid=(nb,),
        in_specs=[
            pl.BlockSpec((rows_per_block, fin), lambda i: (i, 0)),
            pl.BlockSpec((fin, fout), lambda i: (0, 0)),
            pl.BlockSpec((fout,), lambda i: (0,)),
        ],
        out_specs=pl.BlockSpec((rows_per_block, fout), lambda i: (i, 0)),
        out_shape=jax.ShapeDtypeStruct((n, fout), jnp.float32),
    )(h, W, b)


def _bn_relu(z, g, be):
    m = jnp.mean(z, axis=0)
    v = jnp.var(z, axis=0)
    return jax.nn.relu((z - m) / jnp.sqrt(v + 1e-5) * g + be)


def _bn_relu_sync(z, g, be):
    eye = jnp.eye(z.shape[1], dtype=jnp.float32)
    zd = lax.dot_general(z, eye, (((1,), (0,)), ((), ())),
                         precision=lax.Precision.HIGHEST,
                         preferred_element_type=jnp.float32)
    return _bn_relu(zd, g, be)


def _cloud_max_body(h_ref, out_ref):
    out_ref[0] = jnp.max(h_ref[0], axis=0, keepdims=True)


def _cloud_max(h, F):
    out = pl.pallas_call(
        _cloud_max_body,
        grid=(B,),
        in_specs=[pl.BlockSpec((1, P, F), lambda c: (c, 0, 0))],
        out_specs=pl.BlockSpec((1, 1, F), lambda c: (c, 0, 0)),
        out_shape=jax.ShapeDtypeStruct((B, 1, F), jnp.float32),
    )(h.reshape(B, P, F))
    return out.reshape(B, F)


def kernel(x, batch, W1, b1, g1, be1, W2, b2, g2, be2, W3, b3, g3, be3,
           Wg1, bg1, gg1, beg1, Wg2, bg2, gg2, beg2,
           Wb1, bb1, gb1, beb1, Wb2, bb2):
    x3 = x.reshape(B, P, 3)
    h0 = _knn_cov(x3).reshape(N, 12)
    h = _bn_relu_sync(_matmul(h0, W1, b1), g1, be1)
    h = _bn_relu_sync(_matmul(h, W2, b2), g2, be2)
    h = _bn_relu_sync(_matmul(h, W3, b3), g3, be3)
    p1 = _knn_pool(h, 64)
    h = _bn_relu_sync(_matmul(p1, Wg1, bg1), gg1, beg1)
    p2 = _knn_pool(h, 128)
    h = _bn_relu(_matmul(p2, Wg2, bg2), gg2, beg2)
    q = _cloud_max(h, 512)
    q = _bn_relu(_matmul(q, Wb1, bb1), gb1, beb1)
    return _matmul(q, Wb2, bb2)


# speedup vs baseline: 4.7159x; 1.6296x over previous
import functools

import jax
import jax.numpy as jnp
from jax import lax
from jax.experimental import pallas as pl
from jax.experimental.pallas import tpu as pltpu
from jax.experimental.pallas import tpu_sc as plsc

B = 8
P = 2048
K = 16
RB = 256
NRB = P // RB
N = B * P

_BIG = 1e10
_INF = float("inf")
_NEG = -3.0e38
_PREC = None
_GPREC = lax.Precision.HIGHEST


def _topk_selectors(dist):
    col = lax.broadcasted_iota(jnp.int32, dist.shape, 1)
    d = dist
    for _ in range(K):
        m = jnp.min(d, axis=1, keepdims=True)
        cand = jnp.where(d == m, col, P)
        a = jnp.min(cand, axis=1, keepdims=True)
        sel = col == a
        d = jnp.where(sel, _INF, d)
        yield sel.astype(jnp.float32)


def _dist_block(hr, hc, r):
    d2c = jnp.sum(hc * hc, axis=1)
    d2r = jnp.sum(hr * hr, axis=1)
    g = lax.dot_general(hr, hc, (((1,), (1,)), ((), ())),
                        preferred_element_type=jnp.float32,
                        precision=_PREC)
    dist = d2r[:, None] + d2c[None, :] - 2.0 * g
    row = lax.broadcasted_iota(jnp.int32, (RB, P), 0)
    col = lax.broadcasted_iota(jnp.int32, (RB, P), 1)
    return jnp.where(col == row + r * RB, dist + _BIG, dist)


def _tree_sum(terms):
    while len(terms) > 1:
        terms = [terms[i] + terms[i + 1] for i in range(0, len(terms), 2)]
    return terms[0]


def _sublane_sum(terms):
    t = [terms[k] + terms[k + 8] for k in range(8)]
    t = [t[k] + t[k + 4] for k in range(4)]
    t = [t[k] + t[k + 2] for k in range(2)]
    return t[0] + t[1]


def _knn_cov_body(xc_ref, xr_ref, out_ref):
    r = pl.program_id(1)
    xc = xc_ref[0]
    xr = xr_ref[0]
    dist = _dist_block(xr, xc, r)
    nbs = [lax.dot_general(sel, xc, (((1,), (0,)), ((), ())),
                           preferred_element_type=jnp.float32,
                           precision=_GPREC)
           for sel in _topk_selectors(dist)]
    mean = _sublane_sum(nbs) * (1.0 / K)
    cs = [(nb - mean).astype(jnp.bfloat16).astype(jnp.float32) for nb in nbs]
    covs = []
    for i in range(3):
        for j in range(3):
            acc = _tree_sum([ck[:, i:i + 1] * ck[:, j:j + 1] for ck in cs])
            covs.append(acc * (1.0 / K))
    out_ref[0] = jnp.concatenate([xr] + covs, axis=1)


def _knn_cov(x3):
    return pl.pallas_call(
        _knn_cov_body,
        grid=(B, NRB),
        in_specs=[
            pl.BlockSpec((1, P, 3), lambda c, r: (c, 0, 0)),
            pl.BlockSpec((1, RB, 3), lambda c, r: (c, r, 0)),
        ],
        out_specs=pl.BlockSpec((1, RB, 12), lambda c, r: (c, r, 0)),
        out_shape=jax.ShapeDtypeStruct((B, P, 12), jnp.float32),
    )(x3, x3)


def _knn_pool_body(hc_ref, hr_ref, out_ref):
    r = pl.program_id(1)
    hc = hc_ref[0]
    hr = hr_ref[0]
    dist = _dist_block(hr, hc, r)
    acc = jnp.full((RB, hc.shape[1]), _NEG, jnp.float32)
    for sel in _topk_selectors(dist):
        nb = lax.dot_general(sel, hc, (((1,), (0,)), ((), ())),
                             preferred_element_type=jnp.float32,
                             precision=_GPREC)
        acc = jnp.maximum(acc, nb)
    out_ref[0] = acc


def _knn_idx_body(hc_ref, hr_ref, out_ref, tab_ref):
    r = pl.program_id(1)
    c = pl.program_id(0)
    hc = hc_ref[0]
    hr = hr_ref[0]
    dist = _dist_block(hr, hc, r)
    col = lax.broadcasted_iota(jnp.int32, dist.shape, 1)
    d = dist
    outs = []
    for _ in range(K):
        m = jnp.min(d, axis=1, keepdims=True)
        cand = jnp.where(d == m, col, P)
        a = jnp.min(cand, axis=1, keepdims=True)
        d = jnp.where(col == a, _INF, d)
        outs.append(a + c * P)
    out_ref[0] = jnp.concatenate(outs, axis=1)
    F = hr.shape[1]
    if F < 128:
        tab_ref[0] = jnp.concatenate(
            [hr, jnp.zeros((hr.shape[0], 128 - F), jnp.float32)], axis=1)
    else:
        tab_ref[0] = hr


def _knn_idx(hc):
    idx, tab = pl.pallas_call(
        _knn_idx_body,
        grid=(B, NRB),
        in_specs=[
            pl.BlockSpec((1, P, hc.shape[2]), lambda c, r: (c, 0, 0)),
            pl.BlockSpec((1, RB, hc.shape[2]), lambda c, r: (c, r, 0)),
        ],
        out_specs=[
            pl.BlockSpec((1, RB, K), lambda c, r: (c, r, 0)),
            pl.BlockSpec((1, RB, 128), lambda c, r: (c, r, 0)),
        ],
        out_shape=[
            jax.ShapeDtypeStruct((B, P, K), jnp.int32),
            jax.ShapeDtypeStruct((B, P, 128), jnp.float32),
        ],
    )(hc, hc)
    return idx, tab.reshape(N, 128)


_NW = 32
_PTS_W = N // _NW
_GRP = 8
_NGRP = _PTS_W // _GRP


def _sc_pool_body(F, table_hbm, idx_hbm, out_hbm, idx_v, rows_v, out_v, sem):
    wid = lax.axis_index("s") * 2 + lax.axis_index("c")
    pltpu.sync_copy(idx_hbm.at[wid], idx_v)

    def group(g, _):
        pltpu.async_copy(table_hbm.at[idx_v.at[g]], rows_v, sem).wait()

        def point(p, _):
            row = g * _GRP + p
            for c in range(F // 16):
                sl = pl.ds(c * 16, 16)
                acc = rows_v[p * K, sl]
                for k in range(1, K):
                    acc = jnp.maximum(acc, rows_v[p * K + k, sl])
                out_v[row, sl] = acc
            return 0

        return lax.fori_loop(0, _GRP, point, 0)

    lax.fori_loop(0, _NGRP, group, 0)
    pltpu.sync_copy(out_v, out_hbm.at[pl.ds(wid * _PTS_W, _PTS_W)])


def _sc_pool(table, idx, F):
    Fp = 128
    idx3 = idx.reshape(_NW, _NGRP, _GRP * K)
    mesh = plsc.VectorSubcoreMesh(core_axis_name="c", subcore_axis_name="s")
    kfn = functools.partial(
        pl.kernel,
        mesh=mesh,
        out_type=jax.ShapeDtypeStruct((N, F), jnp.float32),
        scratch_types=[
            pltpu.VMEM((_NGRP, _GRP * K), jnp.int32),
            pltpu.VMEM((_GRP * K, Fp), jnp.float32),
            pltpu.VMEM((_PTS_W, F), jnp.float32),
            pltpu.SemaphoreType.DMA,
        ],
    )(functools.partial(_sc_pool_body, F))
    return kfn(table, idx3)


def _knn_pool(h, F):
    hc = h.reshape(B, P, F)
    out = pl.pallas_call(
        _knn_pool_body,
        grid=(B, NRB),
        in_specs=[
            pl.BlockSpec((1, P, F), lambda c, r: (c, 0, 0)),
            pl.BlockSpec((1, RB, F), lambda c, r: (c, r, 0)),
        ],
        out_specs=pl.BlockSpec((1, RB, F), lambda c, r: (c, r, 0)),
        out_shape=jax.ShapeDtypeStruct((B, P, F), jnp.float32),
    )(hc, hc)
    return out.reshape(N, F)


def _matmul_body(h_ref, W_ref, b_ref, out_ref):
    out_ref[...] = jnp.dot(h_ref[...], W_ref[...],
                           preferred_element_type=jnp.float32,
                           precision=_PREC) + b_ref[...][None, :]


def _matmul(h, W, b, rows_per_block=2048):
    n, fin = h.shape
    fout = W.shape[1]
    if n <= rows_per_block:
        return pl.pallas_call(
            _matmul_body,
            out_shape=jax.ShapeDtypeStruct((n, fout), jnp.float32),
        )(h, W, b)
    nb = n // rows_per_block
    return pl.pallas_call(
        _matmul_body,
        grid=(nb,),
        in_specs=[
            pl.BlockSpec((rows_per_block, fin), lambda i: (i, 0)),
            pl.BlockSpec((fin, fout), lambda i: (0, 0)),
            pl.BlockSpec((fout,), lambda i: (0,)),
        ],
        out_specs=pl.BlockSpec((rows_per_block, fout), lambda i: (i, 0)),
        out_shape=jax.ShapeDtypeStruct((n, fout), jnp.float32),
    )(h, W, b)


def _bn_relu(z, g, be):
    m = jnp.mean(z, axis=0)
    v = jnp.var(z, axis=0)
    return jax.nn.relu((z - m) / jnp.sqrt(v + 1e-5) * g + be)


def _bn_relu_sync(z, g, be):
    eye = jnp.eye(z.shape[1], dtype=jnp.float32)
    zd = lax.dot_general(z, eye, (((1,), (0,)), ((), ())),
                         precision=lax.Precision.HIGHEST,
                         preferred_element_type=jnp.float32)
    return _bn_relu(zd, g, be)


def _cloud_max_body(h_ref, out_ref):
    out_ref[0] = jnp.max(h_ref[0], axis=0, keepdims=True)


def _cloud_max(h, F):
    out = pl.pallas_call(
        _cloud_max_body,
        grid=(B,),
        in_specs=[pl.BlockSpec((1, P, F), lambda c: (c, 0, 0))],
        out_specs=pl.BlockSpec((1, 1, F), lambda c: (c, 0, 0)),
        out_shape=jax.ShapeDtypeStruct((B, 1, F), jnp.float32),
    )(h.reshape(B, P, F))
    return out.reshape(B, F)


def kernel(x, batch, W1, b1, g1, be1, W2, b2, g2, be2, W3, b3, g3, be3,
           Wg1, bg1, gg1, beg1, Wg2, bg2, gg2, beg2,
           Wb1, bb1, gb1, beb1, Wb2, bb2):
    x3 = x.reshape(B, P, 3)
    h0 = _knn_cov(x3).reshape(N, 12)
    h = _bn_relu_sync(_matmul(h0, W1, b1), g1, be1)
    h = _bn_relu_sync(_matmul(h, W2, b2), g2, be2)
    h = _bn_relu_sync(_matmul(h, W3, b3), g3, be3)
    idx2, tab2 = _knn_idx(h.reshape(B, P, 64))
    p1 = _sc_pool(tab2, idx2, 64)
    h = _bn_relu_sync(_matmul(p1, Wg1, bg1), gg1, beg1)
    idx3, tab3 = _knn_idx(h.reshape(B, P, 128))
    p2 = _sc_pool(tab3, idx3, 128)
    h = _bn_relu(_matmul(p2, Wg2, bg2), gg2, beg2)
    q = _cloud_max(h, 512)
    q = _bn_relu(_matmul(q, Wb1, bb1), gb1, beb1)
    return _matmul(q, Wb2, bb2)
